# baseline (speedup 1.0000x reference)
import functools

import jax
import jax.numpy as jnp
from jax import lax
from jax.experimental import pallas as pl
from jax.experimental.pallas import tpu as pltpu
from jax.experimental.pallas import tpu_sc as plsc

B = 4096
L = 200
D = 32
V = 1000000
TOT = B * L
NC = 2
NS = 16
NW = NC * NS
PER_W = TOT // NW
CHUNK = 1600
NCHUNK = PER_W // CHUNK

TBLK = 128
NBLK = V // TBLK
VTAIL = NBLK * TBLK
BLK_PER_W = (NBLK + NW - 1) // NW

_mesh = plsc.VectorSubcoreMesh(core_axis_name="c", subcore_axis_name="s")


@functools.partial(
    pl.kernel,
    mesh=_mesh,
    compiler_params=pltpu.CompilerParams(use_tc_tiling_on_sc=True,
                                         needs_layout_passes=False,
                                         disable_bounds_checks=True),
    out_type=jax.ShapeDtypeStruct((V * D,), jnp.float32),
    scratch_types=[
        pltpu.VMEM((D, TBLK), jnp.float32),
        pltpu.VMEM((TBLK * D,), jnp.float32),
    ],
)
def _transpose_sc(tableT_hbm, tail_hbm, out_hbm, inbuf, outbuf):
    wid = lax.axis_index("s") * NC + lax.axis_index("c")
    rows_lo = lax.iota(jnp.int32, 16)
    rows_hi = rows_lo + 16

    def block_body(k, _):
        blk = k * NW + wid

        @pl.when(blk < NBLK)
        def _():
            v0 = blk * TBLK
            pltpu.sync_copy(tableT_hbm.at[:, pl.ds(v0, TBLK)], inbuf)

            def col_body(v8, _):
                for u in range(16):
                    vv = v8 * 16 + u
                    cols = jnp.full((16,), vv, jnp.int32)
                    g1 = plsc.load_gather(inbuf, [rows_lo, cols])
                    g2 = plsc.load_gather(inbuf, [rows_hi, cols])
                    outbuf[pl.ds(vv * D, 16)] = g1
                    outbuf[pl.ds(vv * D + 16, 16)] = g2
                return 0

            lax.fori_loop(0, TBLK // 16, col_body, 0)
            pltpu.sync_copy(outbuf, out_hbm.at[pl.ds(v0 * D, TBLK * D)])

        return 0

    lax.fori_loop(0, BLK_PER_W, block_body, 0)

    @pl.when(wid == 0)
    def _():
        pltpu.sync_copy(tail_hbm, out_hbm.at[pl.ds(VTAIL * D, (V - VTAIL) * D)])


@functools.partial(
    pl.kernel,
    mesh=_mesh,
    compiler_params=pltpu.CompilerParams(use_tc_tiling_on_sc=False),
    out_type=jax.ShapeDtypeStruct((TOT, D), jnp.float32),
    scratch_types=[
        pltpu.VMEM((CHUNK,), jnp.int32),
        pltpu.VMEM((CHUNK,), jnp.int32),
        pltpu.VMEM((CHUNK, D), jnp.float32),
        pltpu.VMEM((CHUNK, D), jnp.float32),
        pltpu.SemaphoreType.DMA,
        pltpu.SemaphoreType.DMA,
    ],
)
def _gather_sc(idx_hbm, table_hbm, out_hbm, idx0, idx1, rows0, rows1,
               gsem0, gsem1):
    wid = lax.axis_index("s") * NC + lax.axis_index("c")
    base = wid * PER_W

    bufs = [(idx0, rows0, gsem0), (idx1, rows1, gsem1)]

    pltpu.sync_copy(idx_hbm.at[pl.ds(base, CHUNK)], idx0)
    pltpu.make_async_copy(table_hbm.at[idx0], rows0, gsem0).start()
    for i in range(NCHUNK):
        cidx, crows, csem = bufs[i % 2]
        nidx, nrows, nsem = bufs[(i + 1) % 2]
        if i + 1 < NCHUNK:
            pltpu.sync_copy(idx_hbm.at[pl.ds(base + (i + 1) * CHUNK, CHUNK)],
                            nidx)
            pltpu.make_async_copy(table_hbm.at[nidx], nrows, nsem).start()
        pltpu.make_async_copy(table_hbm.at[cidx], crows, csem).wait()
        pltpu.sync_copy(crows, out_hbm.at[pl.ds(base + i * CHUNK, CHUNK)])


def _mask_body(ids_ref, mask_ref):
    mask_ref[...] = ids_ref[...] == 0


_mask_tc = pl.pallas_call(
    _mask_body,
    out_shape=jax.ShapeDtypeStruct((B, L), jnp.bool_),
)


def kernel(lookup_ids, embedding_matrix):
    flat_ids = lookup_ids.reshape(-1)
    table_t = embedding_matrix.T
    tail = embedding_matrix[VTAIL:, :].reshape(-1)
    table_lin = _transpose_sc(table_t, tail)
    table_rm = table_lin.reshape(V, D)
    gathered = _gather_sc(flat_ids, table_rm)
    matrices = gathered.reshape(B, L, D)
    mask = _mask_tc(lookup_ids)
    return matrices, mask, lookup_ids


# speedup vs baseline: 1.0905x; 1.0905x over previous
import functools

import jax
import jax.numpy as jnp
from jax import lax
from jax.experimental import pallas as pl
from jax.experimental.pallas import tpu as pltpu
from jax.experimental.pallas import tpu_sc as plsc

B = 4096
L = 200
D = 32
V = 1000000
TOT = B * L
NC = 2
NS = 16
NW = NC * NS
PER_W = TOT // NW
CHUNK = 1600
NCHUNK = PER_W // CHUNK

TBLK = 1024
NBLK = V // TBLK
VTAIL = NBLK * TBLK
NPAIR = ((NBLK + NW - 1) // NW + 1) // 2

_mesh = plsc.VectorSubcoreMesh(core_axis_name="c", subcore_axis_name="s")


@functools.partial(
    pl.kernel,
    mesh=_mesh,
    compiler_params=pltpu.CompilerParams(use_tc_tiling_on_sc=True,
                                         needs_layout_passes=False,
                                         disable_bounds_checks=True),
    out_type=jax.ShapeDtypeStruct((V * D,), jnp.float32),
    scratch_types=[
        pltpu.VMEM((D, TBLK), jnp.float32),
        pltpu.VMEM((D, TBLK), jnp.float32),
        pltpu.VMEM((TBLK * D,), jnp.float32),
        pltpu.SemaphoreType.DMA,
        pltpu.SemaphoreType.DMA,
    ],
)
def _transpose_sc(tableT_hbm, tail_hbm, out_hbm, in0, in1, outbuf, sem0,
                  sem1):
    wid = lax.axis_index("s") * NC + lax.axis_index("c")
    rows_lo = lax.iota(jnp.int32, 16)
    rows_hi = rows_lo + 16

    def start_in(slot, buf, sem):
        blk = slot * NW + wid

        @pl.when(blk < NBLK)
        def _():
            pltpu.make_async_copy(
                tableT_hbm.at[:, pl.ds(blk * TBLK, TBLK)], buf, sem).start()

    def transpose_write(slot, buf, sem):
        blk = slot * NW + wid

        @pl.when(blk < NBLK)
        def _():
            pltpu.make_async_copy(
                tableT_hbm.at[:, pl.ds(blk * TBLK, TBLK)], buf, sem).wait()

            def col_body(v16, _):
                for u in range(16):
                    vv = v16 * 16 + u
                    cols = jnp.full((16,), vv, jnp.int32)
                    outbuf[pl.ds(vv * D, 16)] = plsc.load_gather(
                        buf, [rows_lo, cols])
                    outbuf[pl.ds(vv * D + 16, 16)] = plsc.load_gather(
                        buf, [rows_hi, cols])
                return 0

            lax.fori_loop(0, TBLK // 16, col_body, 0)
            pltpu.sync_copy(outbuf, out_hbm.at[pl.ds(blk * TBLK * D,
                                                     TBLK * D)])

    start_in(0, in0, sem0)

    def pair(k, _):
        s0 = 2 * k
        start_in(s0 + 1, in1, sem1)
        transpose_write(s0, in0, sem0)
        start_in(s0 + 2, in0, sem0)
        transpose_write(s0 + 1, in1, sem1)
        return 0

    lax.fori_loop(0, NPAIR, pair, 0)

    @pl.when(wid == 0)
    def _():
        pltpu.sync_copy(tail_hbm, out_hbm.at[pl.ds(VTAIL * D, (V - VTAIL) * D)])


@functools.partial(
    pl.kernel,
    mesh=_mesh,
    compiler_params=pltpu.CompilerParams(use_tc_tiling_on_sc=False),
    out_type=jax.ShapeDtypeStruct((TOT, D), jnp.float32),
    scratch_types=[
        pltpu.VMEM((CHUNK,), jnp.int32),
        pltpu.VMEM((CHUNK,), jnp.int32),
        pltpu.VMEM((CHUNK, D), jnp.float32),
        pltpu.VMEM((CHUNK, D), jnp.float32),
        pltpu.SemaphoreType.DMA,
        pltpu.SemaphoreType.DMA,
    ],
)
def _gather_sc(idx_hbm, table_hbm, out_hbm, idx0, idx1, rows0, rows1,
               gsem0, gsem1):
    wid = lax.axis_index("s") * NC + lax.axis_index("c")
    base = wid * PER_W

    bufs = [(idx0, rows0, gsem0), (idx1, rows1, gsem1)]

    pltpu.sync_copy(idx_hbm.at[pl.ds(base, CHUNK)], idx0)
    pltpu.make_async_copy(table_hbm.at[idx0], rows0, gsem0).start()
    for i in range(NCHUNK):
        cidx, crows, csem = bufs[i % 2]
        nidx, nrows, nsem = bufs[(i + 1) % 2]
        if i + 1 < NCHUNK:
            pltpu.sync_copy(idx_hbm.at[pl.ds(base + (i + 1) * CHUNK, CHUNK)],
                            nidx)
            pltpu.make_async_copy(table_hbm.at[nidx], nrows, nsem).start()
        pltpu.make_async_copy(table_hbm.at[cidx], crows, csem).wait()
        pltpu.sync_copy(crows, out_hbm.at[pl.ds(base + i * CHUNK, CHUNK)])


def _mask_body(ids_ref, mask_ref):
    mask_ref[...] = ids_ref[...] == 0


_mask_tc = pl.pallas_call(
    _mask_body,
    out_shape=jax.ShapeDtypeStruct((B, L), jnp.bool_),
)


def kernel(lookup_ids, embedding_matrix):
    flat_ids = lookup_ids.reshape(-1)
    table_t = embedding_matrix.T
    tail = embedding_matrix[VTAIL:, :].reshape(-1)
    table_lin = _transpose_sc(table_t, tail)
    table_rm = table_lin.reshape(V, D)
    gathered = _gather_sc(flat_ids, table_rm)
    matrices = gathered.reshape(B, L, D)
    mask = _mask_tc(lookup_ids)
    return matrices, mask, lookup_ids


# speedup vs baseline: 1.5878x; 1.4560x over previous
import functools

import jax
import jax.numpy as jnp
from jax import lax
from jax.experimental import pallas as pl
from jax.experimental.pallas import tpu as pltpu
from jax.experimental.pallas import tpu_sc as plsc

B = 4096
L = 200
D = 32
V = 1000000
TOT = B * L
NC = 2
NS = 16
NW = NC * NS
PER_W = TOT // NW
CHUNK = 1600
NCHUNK = PER_W // CHUNK

TBLK = 1024
NBLK = V // TBLK
VTAIL = NBLK * TBLK
NPAIR = ((NBLK + NW - 1) // NW + 1) // 2

_mesh = plsc.VectorSubcoreMesh(core_axis_name="c", subcore_axis_name="s")


@functools.partial(
    pl.kernel,
    mesh=_mesh,
    compiler_params=pltpu.CompilerParams(use_tc_tiling_on_sc=True,
                                         needs_layout_passes=False,
                                         disable_bounds_checks=True),
    out_type=jax.ShapeDtypeStruct((V * D,), jnp.float32),
    scratch_types=[
        pltpu.VMEM((D, TBLK), jnp.float32),
        pltpu.VMEM((D, TBLK), jnp.float32),
        pltpu.VMEM((TBLK * D,), jnp.float32),
        pltpu.VMEM((D * 17,), jnp.float32),
        pltpu.SemaphoreType.DMA,
        pltpu.SemaphoreType.DMA,
    ],
)
def _transpose_sc(tableT_hbm, tail_hbm, out_hbm, in0, in1, outbuf, stage,
                  sem0, sem1):
    wid = lax.axis_index("s") * NC + lax.axis_index("c")
    iota16 = lax.iota(jnp.int32, 16)
    idx_lo = iota16 * 17
    idx_hi = (iota16 + 16) * 17

    def start_in(slot, buf, sem):
        blk = slot * NW + wid

        @pl.when(blk < NBLK)
        def _():
            pltpu.make_async_copy(
                tableT_hbm.at[:, pl.ds(blk * TBLK, TBLK)], buf, sem).start()

    def transpose_write(slot, buf, sem):
        blk = slot * NW + wid

        @pl.when(blk < NBLK)
        def _():
            pltpu.make_async_copy(
                tableT_hbm.at[:, pl.ds(blk * TBLK, TBLK)], buf, sem).wait()

            def col_body(ci, _):
                c = ci * 16
                for d in range(D):
                    stage[pl.ds(d * 17, 16)] = buf[d, pl.ds(c, 16)]
                for j in range(16):
                    lo = plsc.load_gather(stage, [idx_lo + j])
                    hi = plsc.load_gather(stage, [idx_hi + j])
                    base = (c + j) * D
                    outbuf[pl.ds(base, 16)] = lo
                    outbuf[pl.ds(base + 16, 16)] = hi
                return 0

            lax.fori_loop(0, TBLK // 16, col_body, 0)
            pltpu.sync_copy(outbuf, out_hbm.at[pl.ds(blk * TBLK * D,
                                                     TBLK * D)])

    start_in(0, in0, sem0)

    def pair(k, _):
        s0 = 2 * k
        start_in(s0 + 1, in1, sem1)
        transpose_write(s0, in0, sem0)
        start_in(s0 + 2, in0, sem0)
        transpose_write(s0 + 1, in1, sem1)
        return 0

    lax.fori_loop(0, NPAIR, pair, 0)

    @pl.when(wid == 0)
    def _():
        pltpu.sync_copy(tail_hbm, out_hbm.at[pl.ds(VTAIL * D, (V - VTAIL) * D)])


@functools.partial(
    pl.kernel,
    mesh=_mesh,
    compiler_params=pltpu.CompilerParams(use_tc_tiling_on_sc=False),
    out_type=jax.ShapeDtypeStruct((TOT, D), jnp.float32),
    scratch_types=[
        pltpu.VMEM((CHUNK,), jnp.int32),
        pltpu.VMEM((CHUNK,), jnp.int32),
        pltpu.VMEM((CHUNK, D), jnp.float32),
        pltpu.VMEM((CHUNK, D), jnp.float32),
        pltpu.SemaphoreType.DMA,
        pltpu.SemaphoreType.DMA,
    ],
)
def _gather_sc(idx_hbm, table_hbm, out_hbm, idx0, idx1, rows0, rows1,
               gsem0, gsem1):
    wid = lax.axis_index("s") * NC + lax.axis_index("c")
    base = wid * PER_W

    bufs = [(idx0, rows0, gsem0), (idx1, rows1, gsem1)]

    pltpu.sync_copy(idx_hbm.at[pl.ds(base, CHUNK)], idx0)
    pltpu.make_async_copy(table_hbm.at[idx0], rows0, gsem0).start()
    for i in range(NCHUNK):
        cidx, crows, csem = bufs[i % 2]
        nidx, nrows, nsem = bufs[(i + 1) % 2]
        if i + 1 < NCHUNK:
            pltpu.sync_copy(idx_hbm.at[pl.ds(base + (i + 1) * CHUNK, CHUNK)],
                            nidx)
            pltpu.make_async_copy(table_hbm.at[nidx], nrows, nsem).start()
        pltpu.make_async_copy(table_hbm.at[cidx], crows, csem).wait()
        pltpu.sync_copy(crows, out_hbm.at[pl.ds(base + i * CHUNK, CHUNK)])


def _mask_body(ids_ref, mask_ref):
    mask_ref[...] = ids_ref[...] == 0


_mask_tc = pl.pallas_call(
    _mask_body,
    out_shape=jax.ShapeDtypeStruct((B, L), jnp.bool_),
)


def kernel(lookup_ids, embedding_matrix):
    flat_ids = lookup_ids.reshape(-1)
    table_t = embedding_matrix.T
    tail = embedding_matrix[VTAIL:, :].reshape(-1)
    table_lin = _transpose_sc(table_t, tail)
    table_rm = table_lin.reshape(V, D)
    gathered = _gather_sc(flat_ids, table_rm)
    matrices = gathered.reshape(B, L, D)
    mask = _mask_tc(lookup_ids)
    return matrices, mask, lookup_ids


# speedup vs baseline: 1.6679x; 1.0505x over previous
import functools

import jax
import jax.numpy as jnp
from jax import lax
from jax.experimental import pallas as pl
from jax.experimental.pallas import tpu as pltpu
from jax.experimental.pallas import tpu_sc as plsc

B = 4096
L = 200
D = 32
V = 1000000
TOT = B * L
NC = 2
NS = 16
NW = NC * NS
PER_W = TOT // NW
CHUNK = 1600
NCHUNK = PER_W // CHUNK

TBLK = 896
NBLK = V // TBLK
VTAIL = NBLK * TBLK
NPAIR = ((NBLK + NW - 1) // NW + 1) // 2

_mesh = plsc.VectorSubcoreMesh(core_axis_name="c", subcore_axis_name="s")


@functools.partial(
    pl.kernel,
    mesh=_mesh,
    compiler_params=pltpu.CompilerParams(use_tc_tiling_on_sc=True,
                                         needs_layout_passes=False,
                                         disable_bounds_checks=True),
    out_type=jax.ShapeDtypeStruct((V * D,), jnp.float32),
    scratch_types=[
        pltpu.VMEM((D, TBLK), jnp.float32),
        pltpu.VMEM((D, TBLK), jnp.float32),
        pltpu.VMEM((TBLK * D,), jnp.float32),
        pltpu.VMEM((TBLK * D,), jnp.float32),
        pltpu.VMEM((D * 17,), jnp.float32),
        pltpu.SemaphoreType.DMA,
        pltpu.SemaphoreType.DMA,
        pltpu.SemaphoreType.DMA,
        pltpu.SemaphoreType.DMA,
    ],
)
def _transpose_sc(tableT_hbm, tail_hbm, out_hbm, in0, in1, out0, out1, stage,
                  sem0, sem1, osem0, osem1):
    wid = lax.axis_index("s") * NC + lax.axis_index("c")
    iota16 = lax.iota(jnp.int32, 16)
    idx_lo = iota16 * 17
    idx_hi = (iota16 + 16) * 17

    def start_in(slot, buf, sem):
        blk = slot * NW + wid

        @pl.when(blk < NBLK)
        def _():
            pltpu.make_async_copy(
                tableT_hbm.at[:, pl.ds(blk * TBLK, TBLK)], buf, sem).start()

    def out_copy(slot, obuf, osem):
        blk = slot * NW + wid
        return pltpu.make_async_copy(
            obuf, out_hbm.at[pl.ds(blk * TBLK * D, TBLK * D)], osem)

    def transpose_write(slot, buf, sem, obuf, osem):
        blk = slot * NW + wid

        @pl.when(blk < NBLK)
        def _():
            pltpu.make_async_copy(
                tableT_hbm.at[:, pl.ds(blk * TBLK, TBLK)], buf, sem).wait()

            @pl.when(slot >= 2)
            def _():
                out_copy(slot - 2, obuf, osem).wait()

            def col_body(ci, _):
                c = ci * 16
                for d in range(D):
                    stage[pl.ds(d * 17, 16)] = buf[d, pl.ds(c, 16)]
                for j in range(16):
                    lo = plsc.load_gather(stage, [idx_lo + j])
                    hi = plsc.load_gather(stage, [idx_hi + j])
                    base = (c + j) * D
                    outbuf_slice = obuf
                    outbuf_slice[pl.ds(base, 16)] = lo
                    outbuf_slice[pl.ds(base + 16, 16)] = hi
                return 0

            lax.fori_loop(0, TBLK // 16, col_body, 0)
            out_copy(slot, obuf, osem).start()

    start_in(0, in0, sem0)

    def pair(k, _):
        s0 = 2 * k
        start_in(s0 + 1, in1, sem1)
        transpose_write(s0, in0, sem0, out0, osem0)
        start_in(s0 + 2, in0, sem0)
        transpose_write(s0 + 1, in1, sem1, out1, osem1)
        return 0

    lax.fori_loop(0, NPAIR, pair, 0)

    last_valid = (NBLK - 1 - wid) // NW
    for p, obuf, osem in ((0, out0, osem0), (1, out1, osem1)):
        s_p = last_valid - lax.rem(last_valid - p + 2, 2)

        @pl.when(s_p >= 0)
        def _(s_p=s_p, obuf=obuf, osem=osem):
            out_copy(s_p, obuf, osem).wait()

    @pl.when(wid == 0)
    def _():
        pltpu.sync_copy(tail_hbm, out_hbm.at[pl.ds(VTAIL * D, (V - VTAIL) * D)])


@functools.partial(
    pl.kernel,
    mesh=_mesh,
    compiler_params=pltpu.CompilerParams(use_tc_tiling_on_sc=False),
    out_type=jax.ShapeDtypeStruct((TOT, D), jnp.float32),
    scratch_types=[
        pltpu.VMEM((CHUNK,), jnp.int32),
        pltpu.VMEM((CHUNK,), jnp.int32),
        pltpu.VMEM((CHUNK, D), jnp.float32),
        pltpu.VMEM((CHUNK, D), jnp.float32),
        pltpu.SemaphoreType.DMA,
        pltpu.SemaphoreType.DMA,
    ],
)
def _gather_sc(idx_hbm, table_hbm, out_hbm, idx0, idx1, rows0, rows1,
               gsem0, gsem1):
    wid = lax.axis_index("s") * NC + lax.axis_index("c")
    base = wid * PER_W

    bufs = [(idx0, rows0, gsem0), (idx1, rows1, gsem1)]

    pltpu.sync_copy(idx_hbm.at[pl.ds(base, CHUNK)], idx0)
    pltpu.make_async_copy(table_hbm.at[idx0], rows0, gsem0).start()
    for i in range(NCHUNK):
        cidx, crows, csem = bufs[i % 2]
        nidx, nrows, nsem = bufs[(i + 1) % 2]
        if i + 1 < NCHUNK:
            pltpu.sync_copy(idx_hbm.at[pl.ds(base + (i + 1) * CHUNK, CHUNK)],
                            nidx)
            pltpu.make_async_copy(table_hbm.at[nidx], nrows, nsem).start()
        pltpu.make_async_copy(table_hbm.at[cidx], crows, csem).wait()
        pltpu.sync_copy(crows, out_hbm.at[pl.ds(base + i * CHUNK, CHUNK)])


def _mask_body(ids_ref, mask_ref):
    mask_ref[...] = ids_ref[...] == 0


_mask_tc = pl.pallas_call(
    _mask_body,
    out_shape=jax.ShapeDtypeStruct((B, L), jnp.bool_),
)


def kernel(lookup_ids, embedding_matrix):
    flat_ids = lookup_ids.reshape(-1)
    table_t = embedding_matrix.T
    tail = embedding_matrix[VTAIL:, :].reshape(-1)
    table_lin = _transpose_sc(table_t, tail)
    table_rm = table_lin.reshape(V, D)
    gathered = _gather_sc(flat_ids, table_rm)
    matrices = gathered.reshape(B, L, D)
    mask = _mask_tc(lookup_ids)
    return matrices, mask, lookup_ids
